# Initial kernel scaffold; baseline (speedup 1.0000x reference)
import jax
import jax.numpy as jnp
from jax.experimental import pallas as pl


def kernel(x, edge, W1, b1, W2, b2):
    raise NotImplementedError("write your pallas kernel here")


# speedup vs baseline: 22.0473x; 22.0473x over previous
import functools

import jax
import jax.numpy as jnp
from jax import lax
from jax.experimental import pallas as pl
from jax.experimental.pallas import tpu as pltpu
from jax.experimental.pallas import tpu_sc as plsc

N = 10000
E = 320000
NC, NS = 2, 16
NW = NC * NS
K = 128
NCH = 80
EPW = K * NCH
E_PAD = EPW * NW
RPT = 626
NROWS = RPT * NS

_mesh = plsc.VectorSubcoreMesh(core_axis_name="c", subcore_axis_name="s")


def _make_sc_pass(D, with_gather):
  scratch = [
      pltpu.VMEM((NCH, K), jnp.int32),
      pltpu.VMEM((K, D), jnp.float32),
      pltpu.VMEM_SHARED((NROWS, D), jnp.float32),
      pltpu.SemaphoreType.DMA,
  ]
  if with_gather:
    scratch.insert(1, pltpu.VMEM((NCH, K), jnp.int32))

  @functools.partial(
      pl.kernel,
      mesh=_mesh,
      out_type=jax.ShapeDtypeStruct((NW, RPT, D), jnp.float32),
      scratch_types=scratch,
      compiler_params=pltpu.CompilerParams(use_tc_tiling_on_sc=False),
  )
  def sc_pass(*refs):
    if with_gather:
      (table, srcidx, dstidx, zeros, out,
       dst_v, src_v, rows, acc, sem) = refs
    else:
      (dstidx, zeros, ones, out,
       dst_v, rows, acc, sem) = refs

    c = lax.axis_index("c")
    s = lax.axis_index("s")
    wid = c * NS + s

    pltpu.sync_copy(zeros, acc.at[pl.ds(s * RPT, RPT)])
    pltpu.sync_copy(dstidx.at[wid], dst_v)
    if with_gather:
      pltpu.sync_copy(srcidx.at[wid], src_v)
    else:
      pltpu.sync_copy(ones, rows)
    plsc.subcore_barrier()

    def body(j, carry):
      if with_gather:
        pltpu.async_copy(table.at[src_v.at[j]], rows, sem).wait()
      pltpu.sync_copy(rows, acc.at[dst_v.at[j]], add=True)
      return carry

    lax.fori_loop(0, NCH, body, 0)
    plsc.subcore_barrier()

    pltpu.sync_copy(acc.at[pl.ds(s * RPT, RPT)], out.at[wid])

  return sc_pass


_sc_deg = _make_sc_pass(16, with_gather=False)
_sc_l1 = _make_sc_pass(16, with_gather=True)
_sc_l2 = _make_sc_pass(48, with_gather=True)


def _deg_dis(dacc_ref):
  deg = dacc_ref[0, :N, 0] + dacc_ref[1, :N, 0] + 1.0
  return lax.rsqrt(deg)


def _tc_y1(dacc_ref, x_ref, w1_ref, y1_ref):
  dis = _deg_dis(dacc_ref)
  xw = jnp.dot(x_ref[...], w1_ref[...], preferred_element_type=jnp.float32)
  y1_ref[...] = xw * dis[:, None]


def _tc_y2(dacc_ref, acc1_ref, y1_ref, w2_ref, b1_ref, y2_ref):
  dis = _deg_dis(dacc_ref)
  agg = acc1_ref[0, :N, :] + acc1_ref[1, :N, :] + y1_ref[...]
  h = jnp.maximum(dis[:, None] * agg + b1_ref[...], 0.0)
  xw = jnp.dot(h, w2_ref[...], preferred_element_type=jnp.float32)
  y2_ref[...] = jnp.concatenate(
      [xw * dis[:, None], jnp.zeros((N, 8), jnp.float32)], axis=1)


def _tc_out(dacc_ref, acc2_ref, y2_ref, b2_ref, out_ref):
  dis = _deg_dis(dacc_ref)
  agg = acc2_ref[0, :N, :] + acc2_ref[1, :N, :] + y2_ref[...]
  o = (dis[:, None] * agg)[:, :40] + b2_ref[...]
  m = jnp.max(o, axis=1, keepdims=True)
  z = o - m
  lse = jnp.log(jnp.sum(jnp.exp(z), axis=1, keepdims=True))
  out_ref[...] = z - lse


def kernel(x, edge, W1, b1, W2, b2):
  src = edge[0].astype(jnp.int32)
  dst = edge[1].astype(jnp.int32)

  npad = E_PAD - E
  pad_src = jnp.zeros((npad,), jnp.int32)
  pad_dst = N + (jnp.arange(npad, dtype=jnp.int32) % 16)
  src3 = jnp.concatenate([src, pad_src]).reshape(NW, NCH, K)
  dst3 = jnp.concatenate([dst, pad_dst]).reshape(NW, NCH, K)

  zeros16 = jnp.zeros((RPT, 16), jnp.float32)
  zeros48 = jnp.zeros((RPT, 48), jnp.float32)
  ones16 = jnp.ones((K, 16), jnp.float32)

  dacc = _sc_deg(dst3, zeros16, ones16).reshape(NC, NROWS, 16)

  y1 = pl.pallas_call(
      _tc_y1,
      out_shape=jax.ShapeDtypeStruct((N, 16), jnp.float32),
  )(dacc, x, W1)

  acc1 = _sc_l1(y1, src3, dst3, zeros16).reshape(NC, NROWS, 16)

  y2 = pl.pallas_call(
      _tc_y2,
      out_shape=jax.ShapeDtypeStruct((N, 48), jnp.float32),
  )(dacc, acc1, y1, W2, b1)

  acc2 = _sc_l2(y2, src3, dst3, zeros48).reshape(NC, NROWS, 48)

  out = pl.pallas_call(
      _tc_out,
      out_shape=jax.ShapeDtypeStruct((N, 40), jnp.float32),
  )(dacc, acc2, y2, b2)

  return out


# speedup vs baseline: 26.6041x; 1.2067x over previous
import functools

import jax
import jax.numpy as jnp
from jax import lax
from jax.experimental import pallas as pl
from jax.experimental.pallas import tpu as pltpu
from jax.experimental.pallas import tpu_sc as plsc

N = 10000
E = 320000
NC, NS = 2, 16
NW = NC * NS
K = 128
NCH = 80
EPW = K * NCH
E_PAD = EPW * NW
RPT = 626
NROWS = RPT * NS

_mesh = plsc.VectorSubcoreMesh(core_axis_name="c", subcore_axis_name="s")


NBUF = 4
G = NCH // NBUF


def _make_sc_pass(D, with_gather):
  scratch = [
      pltpu.VMEM((NCH, K), jnp.int32),
      pltpu.VMEM_SHARED((NROWS, D), jnp.float32),
  ]
  if with_gather:
    scratch.append(pltpu.VMEM((NCH, K), jnp.int32))
    scratch += [pltpu.VMEM((K, D), jnp.float32) for _ in range(NBUF)]
    scratch += [pltpu.SemaphoreType.DMA for _ in range(2 * NBUF)]
  else:
    scratch.append(pltpu.VMEM((K, D), jnp.float32))
    scratch += [pltpu.SemaphoreType.DMA for _ in range(NBUF)]

  @functools.partial(
      pl.kernel,
      mesh=_mesh,
      out_type=jax.ShapeDtypeStruct((NW * RPT, D), jnp.float32),
      scratch_types=scratch,
      compiler_params=pltpu.CompilerParams(use_tc_tiling_on_sc=False),
  )
  def sc_pass(*refs):
    if with_gather:
      (table, srcidx, dstidx, zeros, out, dst_v, acc, src_v) = refs[:8]
      rows = refs[8:8 + NBUF]
      gsem = refs[8 + NBUF:8 + 2 * NBUF]
      ssem = refs[8 + 2 * NBUF:]
    else:
      (dstidx, zeros, ones, out, dst_v, acc, rows1) = refs[:7]
      ssem = refs[7:]

    c = lax.axis_index("c")
    s = lax.axis_index("s")
    wid = c * NS + s

    pltpu.sync_copy(zeros, acc.at[pl.ds(s * RPT, RPT)])
    pltpu.sync_copy(dstidx.at[wid], dst_v)
    if with_gather:
      pltpu.sync_copy(srcidx.at[wid], src_v)
    else:
      pltpu.sync_copy(ones, rows1)
    plsc.subcore_barrier()

    if with_gather:
      def gather(b, j):
        pltpu.async_copy(table.at[src_v.at[j]], rows[b], gsem[b])

      def gather_wait(b, j):
        pltpu.make_async_copy(table.at[src_v.at[j]], rows[b], gsem[b]).wait()

      def scatter(b, j):
        pltpu.async_copy(rows[b], acc.at[dst_v.at[j]], ssem[b], add=True)

      def scatter_wait(b, j):
        pltpu.make_async_copy(rows[b], acc.at[dst_v.at[j]], ssem[b]).wait()

      for b in range(NBUF):
        gather(b, b)

      def body(g, carry):
        j0 = g * NBUF
        for b in range(NBUF):
          gather_wait(b, j0 + b)
          scatter(b, j0 + b)
        for b in range(NBUF):
          scatter_wait(b, j0 + b)
          gather(b, lax.rem(j0 + NBUF + b, NCH))
        return carry

      lax.fori_loop(0, G, body, 0)
      for b in range(NBUF):
        gather_wait(b, b)
    else:
      def scatter1(b, j):
        pltpu.async_copy(rows1, acc.at[dst_v.at[j]], ssem[b], add=True)

      def scatter1_wait(b, j):
        pltpu.make_async_copy(rows1, acc.at[dst_v.at[j]], ssem[b]).wait()

      def body(g, carry):
        j0 = g * NBUF
        for b in range(NBUF):
          scatter1(b, j0 + b)
        for b in range(NBUF):
          scatter1_wait(b, j0 + b)
        return carry

      lax.fori_loop(0, G, body, 0)

    plsc.subcore_barrier()
    pltpu.sync_copy(acc.at[pl.ds(s * RPT, RPT)],
                    out.at[pl.ds(wid * RPT, RPT)])

  return sc_pass


_sc_deg = _make_sc_pass(16, with_gather=False)
_sc_l1 = _make_sc_pass(16, with_gather=True)
_sc_l2 = _make_sc_pass(48, with_gather=True)


def _deg_dis(dacc_ref):
  deg = dacc_ref[0, :N, 0] + dacc_ref[1, :N, 0] + 1.0
  return lax.rsqrt(deg)


def _tc_y1(dacc_ref, x_ref, w1_ref, y1_ref):
  dis = _deg_dis(dacc_ref)
  xw = jnp.dot(x_ref[...], w1_ref[...], preferred_element_type=jnp.float32)
  y1_ref[...] = xw * dis[:, None]


def _tc_y2(dacc_ref, acc1_ref, y1_ref, w2_ref, b1_ref, y2_ref):
  dis = _deg_dis(dacc_ref)
  agg = acc1_ref[0, :N, :] + acc1_ref[1, :N, :] + y1_ref[...]
  h = jnp.maximum(dis[:, None] * agg + b1_ref[...], 0.0)
  xw = jnp.dot(h, w2_ref[...], preferred_element_type=jnp.float32)
  y2_ref[...] = jnp.concatenate(
      [xw * dis[:, None], jnp.zeros((N, 8), jnp.float32)], axis=1)


def _tc_out(dacc_ref, acc2_ref, y2_ref, b2_ref, out_ref):
  dis = _deg_dis(dacc_ref)
  agg = acc2_ref[0, :N, :] + acc2_ref[1, :N, :] + y2_ref[...]
  o = (dis[:, None] * agg)[:, :40] + b2_ref[...]
  m = jnp.max(o, axis=1, keepdims=True)
  z = o - m
  lse = jnp.log(jnp.sum(jnp.exp(z), axis=1, keepdims=True))
  out_ref[...] = z - lse


def kernel(x, edge, W1, b1, W2, b2):
  src = edge[0].astype(jnp.int32)
  dst = edge[1].astype(jnp.int32)

  npad = E_PAD - E
  pad_src = jnp.zeros((npad,), jnp.int32)
  pad_dst = N + (jnp.arange(npad, dtype=jnp.int32) % 16)
  src3 = jnp.concatenate([src, pad_src]).reshape(NW, NCH, K)
  dst3 = jnp.concatenate([dst, pad_dst]).reshape(NW, NCH, K)

  zeros16 = jnp.zeros((RPT, 16), jnp.float32)
  zeros48 = jnp.zeros((RPT, 48), jnp.float32)
  ones16 = jnp.ones((K, 16), jnp.float32)

  dacc = _sc_deg(dst3, zeros16, ones16).reshape(NC, NROWS, 16)

  y1 = pl.pallas_call(
      _tc_y1,
      out_shape=jax.ShapeDtypeStruct((N, 16), jnp.float32),
  )(dacc, x, W1)

  acc1 = _sc_l1(y1, src3, dst3, zeros16).reshape(NC, NROWS, 16)

  y2 = pl.pallas_call(
      _tc_y2,
      out_shape=jax.ShapeDtypeStruct((N, 48), jnp.float32),
  )(dacc, acc1, y1, W2, b1)

  acc2 = _sc_l2(y2, src3, dst3, zeros48).reshape(NC, NROWS, 48)

  out = pl.pallas_call(
      _tc_out,
      out_shape=jax.ShapeDtypeStruct((N, 40), jnp.float32),
  )(dacc, acc2, y2, b2)

  return out


# speedup vs baseline: 36.8654x; 1.3857x over previous
import functools

import jax
import jax.numpy as jnp
from jax import lax
from jax.experimental import pallas as pl
from jax.experimental.pallas import tpu as pltpu
from jax.experimental.pallas import tpu_sc as plsc

N = 10000
E = 320000
NC, NS = 2, 16
NW = NC * NS
K = 128
NCH = 80
EPW = K * NCH
E_PAD = EPW * NW
RPT = 626
NROWS = RPT * NS

_mesh = plsc.VectorSubcoreMesh(core_axis_name="c", subcore_axis_name="s")


NBUF = 4
G = NCH // NBUF


def _make_sc_pass(D, with_gather):
  scratch = [
      pltpu.VMEM((NCH, K), jnp.int32),
      pltpu.VMEM_SHARED((NROWS, D), jnp.float32),
  ]
  if with_gather:
    scratch.append(pltpu.VMEM((NCH, K), jnp.int32))
    scratch += [pltpu.VMEM((K, D), jnp.float32) for _ in range(NBUF)]
    scratch += [pltpu.SemaphoreType.DMA for _ in range(2 * NBUF)]
  else:
    scratch.append(pltpu.VMEM((K, D), jnp.float32))
    scratch += [pltpu.SemaphoreType.DMA for _ in range(NBUF)]

  @functools.partial(
      pl.kernel,
      mesh=_mesh,
      out_type=jax.ShapeDtypeStruct((NW * RPT, D), jnp.float32),
      scratch_types=scratch,
      compiler_params=pltpu.CompilerParams(use_tc_tiling_on_sc=False),
  )
  def sc_pass(*refs):
    if with_gather:
      (table, srcidx, dstidx, zeros, out, dst_v, acc, src_v) = refs[:8]
      rows = refs[8:8 + NBUF]
      gsem = refs[8 + NBUF:8 + 2 * NBUF]
      ssem = refs[8 + 2 * NBUF:]
    else:
      (dstidx, zeros, ones, out, dst_v, acc, rows1) = refs[:7]
      ssem = refs[7:]

    c = lax.axis_index("c")
    s = lax.axis_index("s")
    wid = c * NS + s

    pltpu.sync_copy(zeros, acc.at[pl.ds(s * RPT, RPT)])
    pltpu.sync_copy(dstidx.at[wid], dst_v)
    if with_gather:
      pltpu.sync_copy(srcidx.at[wid], src_v)
    else:
      pltpu.sync_copy(ones, rows1)
    plsc.subcore_barrier()

    if with_gather:
      def gather(b, j):
        pltpu.async_copy(table.at[src_v.at[j]], rows[b], gsem[b])

      def gather_wait(b, j):
        pltpu.make_async_copy(table.at[src_v.at[j]], rows[b], gsem[b]).wait()

      def scatter(b, j):
        pltpu.async_copy(rows[b], acc.at[dst_v.at[j]], ssem[b], add=True)

      def scatter_wait(b, j):
        pltpu.make_async_copy(rows[b], acc.at[dst_v.at[j]], ssem[b]).wait()

      for b in range(NBUF):
        gather(b, b)

      def body(g, carry):
        j0 = g * NBUF
        for b in range(NBUF):
          gather_wait(b, j0 + b)
          scatter(b, j0 + b)
        for b in range(NBUF):
          scatter_wait(b, j0 + b)
          gather(b, lax.rem(j0 + NBUF + b, NCH))
        return carry

      lax.fori_loop(0, G, body, 0)
      for b in range(NBUF):
        gather_wait(b, b)
    else:
      def scatter1(b, j):
        pltpu.async_copy(rows1, acc.at[dst_v.at[j]], ssem[b], add=True)

      def scatter1_wait(b, j):
        pltpu.make_async_copy(rows1, acc.at[dst_v.at[j]], ssem[b]).wait()

      def body(g, carry):
        j0 = g * NBUF
        for b in range(NBUF):
          scatter1(b, j0 + b)
        for b in range(NBUF):
          scatter1_wait(b, j0 + b)
        return carry

      lax.fori_loop(0, G, body, 0)

    plsc.subcore_barrier()
    pltpu.sync_copy(acc.at[pl.ds(s * RPT, RPT)],
                    out.at[pl.ds(wid * RPT, RPT)])

  return sc_pass


_sc_deg = _make_sc_pass(16, with_gather=False)
_sc_agg = _make_sc_pass(16, with_gather=True)


def _deg_dis(dacc_ref):
  deg = dacc_ref[0, :N, 0] + dacc_ref[1, :N, 0] + 1.0
  return lax.rsqrt(deg)


def _tc_y1(dacc_ref, x_ref, w1_ref, y1_ref):
  dis = _deg_dis(dacc_ref)
  xw = jnp.dot(x_ref[...], w1_ref[...], preferred_element_type=jnp.float32)
  y1_ref[...] = xw * dis[:, None]


def _tc_y2(dacc_ref, acc1_ref, y1_ref, b1_ref, y2_ref):
  dis = _deg_dis(dacc_ref)
  agg = acc1_ref[0, :N, :] + acc1_ref[1, :N, :] + y1_ref[...]
  h = jnp.maximum(dis[:, None] * agg + b1_ref[...], 0.0)
  y2_ref[...] = h * dis[:, None]


def _tc_out(dacc_ref, acc2_ref, y2_ref, w2_ref, b2_ref, out_ref):
  dis = _deg_dis(dacc_ref)
  z = dis[:, None] * (acc2_ref[0, :N, :] + acc2_ref[1, :N, :] + y2_ref[...])
  o = jnp.dot(z, w2_ref[...], preferred_element_type=jnp.float32) + b2_ref[...]
  m = jnp.max(o, axis=1, keepdims=True)
  zz = o - m
  lse = jnp.log(jnp.sum(jnp.exp(zz), axis=1, keepdims=True))
  out_ref[...] = zz - lse


def kernel(x, edge, W1, b1, W2, b2):
  src = edge[0].astype(jnp.int32)
  dst = edge[1].astype(jnp.int32)

  npad = E_PAD - E
  pad_src = jnp.zeros((npad,), jnp.int32)
  pad_dst = N + (jnp.arange(npad, dtype=jnp.int32) % 16)
  src3 = jnp.concatenate([src, pad_src]).reshape(NW, NCH, K)
  dst3 = jnp.concatenate([dst, pad_dst]).reshape(NW, NCH, K)

  zeros16 = jnp.zeros((RPT, 16), jnp.float32)
  ones16 = jnp.ones((K, 16), jnp.float32)

  dacc = _sc_deg(dst3, zeros16, ones16).reshape(NC, NROWS, 16)

  y1 = pl.pallas_call(
      _tc_y1,
      out_shape=jax.ShapeDtypeStruct((N, 16), jnp.float32),
  )(dacc, x, W1)

  acc1 = _sc_agg(y1, src3, dst3, zeros16).reshape(NC, NROWS, 16)

  y2 = pl.pallas_call(
      _tc_y2,
      out_shape=jax.ShapeDtypeStruct((N, 16), jnp.float32),
  )(dacc, acc1, y1, b1)

  acc2 = _sc_agg(y2, src3, dst3, zeros16).reshape(NC, NROWS, 16)

  out = pl.pallas_call(
      _tc_out,
      out_shape=jax.ShapeDtypeStruct((N, 40), jnp.float32),
  )(dacc, acc2, y2, W2, b2)

  return out


# speedup vs baseline: 52.7823x; 1.4318x over previous
import functools

import jax
import jax.numpy as jnp
from jax import lax
from jax.experimental import pallas as pl
from jax.experimental.pallas import tpu as pltpu
from jax.experimental.pallas import tpu_sc as plsc

N = 10000
E = 320000
NC, NS = 2, 16
NW = NC * NS
K = 128
NCH = 80
EPW = K * NCH
E_PAD = EPW * NW
RPT = 626
NROWS = RPT * NS

_mesh = plsc.VectorSubcoreMesh(core_axis_name="c", subcore_axis_name="s")


NBUF = 4
G = NCH // NBUF


def _make_sc_pass(D, with_gather):
  scratch = [
      pltpu.VMEM((NCH, K), jnp.int32),
      pltpu.VMEM_SHARED((NROWS, D), jnp.float32),
  ]
  if with_gather:
    scratch.append(pltpu.VMEM((NCH, K), jnp.int32))
    scratch += [pltpu.VMEM((K, D), jnp.float32) for _ in range(NBUF)]
    scratch += [pltpu.SemaphoreType.DMA for _ in range(2 * NBUF)]
  else:
    scratch.append(pltpu.VMEM((K, D), jnp.float32))
    scratch += [pltpu.SemaphoreType.DMA for _ in range(NBUF)]

  @functools.partial(
      pl.kernel,
      mesh=_mesh,
      out_type=jax.ShapeDtypeStruct((NW * RPT, D), jnp.float32),
      scratch_types=scratch,
      compiler_params=pltpu.CompilerParams(use_tc_tiling_on_sc=False),
  )
  def sc_pass(*refs):
    if with_gather:
      (table, srcidx, dstidx, zeros, out, dst_v, acc, src_v) = refs[:8]
      rows = refs[8:8 + NBUF]
      gsem = refs[8 + NBUF:8 + 2 * NBUF]
      ssem = refs[8 + 2 * NBUF:]
    else:
      (dstidx, zeros, ones, out, dst_v, acc, rows1) = refs[:7]
      ssem = refs[7:]

    c = lax.axis_index("c")
    s = lax.axis_index("s")
    wid = c * NS + s

    pltpu.sync_copy(zeros, acc.at[pl.ds(s * RPT, RPT)])
    pltpu.sync_copy(dstidx.at[wid], dst_v)
    if with_gather:
      pltpu.sync_copy(srcidx.at[wid], src_v)
    else:
      pltpu.sync_copy(ones, rows1)
    plsc.subcore_barrier()

    if with_gather:
      def gather(b, j):
        pltpu.async_copy(table.at[src_v.at[j]], rows[b], gsem[b])

      def gather_wait(b, j):
        pltpu.make_async_copy(table.at[src_v.at[j]], rows[b], gsem[b]).wait()

      def scatter(b, j):
        pltpu.async_copy(rows[b], acc.at[dst_v.at[j]], ssem[b], add=True)

      def scatter_wait(b, j):
        pltpu.make_async_copy(rows[b], acc.at[dst_v.at[j]], ssem[b]).wait()

      for b in range(NBUF):
        gather(b, b)

      def body(g, carry):
        j0 = g * NBUF
        for b in range(NBUF):
          gather_wait(b, j0 + b)
          scatter(b, j0 + b)
        for b in range(NBUF):
          scatter_wait(b, j0 + b)
          gather(b, lax.rem(j0 + NBUF + b, NCH))
        return carry

      lax.fori_loop(0, G, body, 0)
      for b in range(NBUF):
        gather_wait(b, b)
    else:
      def scatter1(b, j):
        pltpu.async_copy(rows1, acc.at[dst_v.at[j]], ssem[b], add=True)

      def scatter1_wait(b, j):
        pltpu.make_async_copy(rows1, acc.at[dst_v.at[j]], ssem[b]).wait()

      def body(g, carry):
        j0 = g * NBUF
        for b in range(NBUF):
          scatter1(b, j0 + b)
        for b in range(NBUF):
          scatter1_wait(b, j0 + b)
        return carry

      lax.fori_loop(0, G, body, 0)

    plsc.subcore_barrier()
    pltpu.sync_copy(acc.at[pl.ds(s * RPT, RPT)],
                    out.at[pl.ds(wid * RPT, RPT)])

  return sc_pass


_sc_deg = _make_sc_pass(16, with_gather=False)
_sc_agg = _make_sc_pass(16, with_gather=True)


def _deg_dis(dacc_ref):
  deg = dacc_ref[:N, 0] + dacc_ref[NROWS:NROWS + N, 0] + 1.0
  return lax.rsqrt(deg)


def _tc_y1(dacc_ref, x_ref, w1_ref, y1_ref):
  dis = _deg_dis(dacc_ref)
  xw = jnp.dot(x_ref[...], w1_ref[...], preferred_element_type=jnp.float32)
  y1_ref[...] = xw * dis[:, None]


def _tc_y2(dacc_ref, acc1_ref, y1_ref, b1_ref, y2_ref):
  dis = _deg_dis(dacc_ref)
  agg = acc1_ref[:N, :] + acc1_ref[NROWS:NROWS + N, :] + y1_ref[...]
  h = jnp.maximum(dis[:, None] * agg + b1_ref[...], 0.0)
  y2_ref[...] = h * dis[:, None]


def _tc_out(dacc_ref, acc2_ref, y2_ref, w2_ref, b2_ref, out_ref):
  dis = _deg_dis(dacc_ref)
  z = dis[:, None] * (acc2_ref[:N, :] + acc2_ref[NROWS:NROWS + N, :] + y2_ref[...])
  o = jnp.dot(z, w2_ref[...], preferred_element_type=jnp.float32) + b2_ref[...]
  m = jnp.max(o, axis=1, keepdims=True)
  zz = o - m
  lse = jnp.log(jnp.sum(jnp.exp(zz), axis=1, keepdims=True))
  out_ref[...] = zz - lse


def kernel(x, edge, W1, b1, W2, b2):
  src = edge[0].astype(jnp.int32)
  dst = edge[1].astype(jnp.int32)

  npad = E_PAD - E
  pad_src = jnp.arange(npad, dtype=jnp.int32) % N
  pad_dst = N + (jnp.arange(npad, dtype=jnp.int32) % 16)
  src3 = jnp.concatenate([src, pad_src]).reshape(NW, NCH, K)
  dst3 = jnp.concatenate([dst, pad_dst]).reshape(NW, NCH, K)

  zeros16 = jnp.zeros((RPT, 16), jnp.float32)
  ones16 = jnp.ones((K, 16), jnp.float32)

  dacc = _sc_deg(dst3, zeros16, ones16)

  y1 = pl.pallas_call(
      _tc_y1,
      out_shape=jax.ShapeDtypeStruct((N, 16), jnp.float32),
  )(dacc, x, W1)

  acc1 = _sc_agg(y1, src3, dst3, zeros16)

  y2 = pl.pallas_call(
      _tc_y2,
      out_shape=jax.ShapeDtypeStruct((N, 16), jnp.float32),
  )(dacc, acc1, y1, b1)

  acc2 = _sc_agg(y2, src3, dst3, zeros16)

  out = pl.pallas_call(
      _tc_out,
      out_shape=jax.ShapeDtypeStruct((N, 40), jnp.float32),
  )(dacc, acc2, y2, W2, b2)

  return out


# speedup vs baseline: 52.8684x; 1.0016x over previous
import functools

import jax
import jax.numpy as jnp
from jax import lax
from jax.experimental import pallas as pl
from jax.experimental.pallas import tpu as pltpu
from jax.experimental.pallas import tpu_sc as plsc

N = 10000
E = 320000
NC, NS = 2, 16
NW = NC * NS
K = 128
NCH = 80
NCHUNKS = E // K
G_LAST = (NCHUNKS - (NW - 1) * NCH) // 4
RPT = 626
NROWS = RPT * NS

_mesh = plsc.VectorSubcoreMesh(core_axis_name="c", subcore_axis_name="s")


NBUF = 4
G = NCH // NBUF


def _make_sc_pass(D, with_gather):
  scratch = [
      pltpu.VMEM((NCH, K), jnp.int32),
      pltpu.VMEM_SHARED((NROWS, D), jnp.float32),
  ]
  if with_gather:
    scratch.append(pltpu.VMEM((NCH, K), jnp.int32))
    scratch += [pltpu.VMEM((K, D), jnp.float32) for _ in range(NBUF)]
    scratch += [pltpu.SemaphoreType.DMA for _ in range(2 * NBUF)]
  else:
    scratch.append(pltpu.VMEM((K, D), jnp.float32))
    scratch += [pltpu.SemaphoreType.DMA for _ in range(NBUF)]

  @functools.partial(
      pl.kernel,
      mesh=_mesh,
      out_type=jax.ShapeDtypeStruct((NW * RPT, D), jnp.float32),
      scratch_types=scratch,
      compiler_params=pltpu.CompilerParams(use_tc_tiling_on_sc=False),
  )
  def sc_pass(*refs):
    if with_gather:
      (table, srcidx, dstidx, zeros, out, dst_v, acc, src_v) = refs[:8]
      rows = refs[8:8 + NBUF]
      gsem = refs[8 + NBUF:8 + 2 * NBUF]
      ssem = refs[8 + 2 * NBUF:]
    else:
      (dstidx, zeros, ones, out, dst_v, acc, rows1) = refs[:7]
      ssem = refs[7:]

    c = lax.axis_index("c")
    s = lax.axis_index("s")
    wid = c * NS + s

    base = jnp.minimum(wid * NCH, NCHUNKS - NCH)
    g0 = jnp.where(wid == NW - 1, G - G_LAST, 0)

    pltpu.sync_copy(zeros, acc.at[pl.ds(s * RPT, RPT)])
    pltpu.sync_copy(dstidx.at[pl.ds(base, NCH)], dst_v)
    if with_gather:
      pltpu.sync_copy(srcidx.at[pl.ds(base, NCH)], src_v)
    else:
      pltpu.sync_copy(ones, rows1)
    plsc.subcore_barrier()

    if with_gather:
      def gather(b, j):
        pltpu.async_copy(table.at[src_v.at[j]], rows[b], gsem[b])

      def gather_wait(b, j):
        pltpu.make_async_copy(table.at[src_v.at[j]], rows[b], gsem[b]).wait()

      def scatter(b, j):
        pltpu.async_copy(rows[b], acc.at[dst_v.at[j]], ssem[b], add=True)

      def scatter_wait(b, j):
        pltpu.make_async_copy(rows[b], acc.at[dst_v.at[j]], ssem[b]).wait()

      for b in range(NBUF):
        gather(b, g0 * NBUF + b)

      def body(g, carry):
        j0 = g * NBUF
        for b in range(NBUF):
          gather_wait(b, j0 + b)
          scatter(b, j0 + b)
        for b in range(NBUF):
          scatter_wait(b, j0 + b)
          gather(b, lax.rem(j0 + NBUF + b, NCH))
        return carry

      lax.fori_loop(g0, G, body, 0)
      for b in range(NBUF):
        gather_wait(b, b)
    else:
      def scatter1(b, j):
        pltpu.async_copy(rows1, acc.at[dst_v.at[j]], ssem[b], add=True)

      def scatter1_wait(b, j):
        pltpu.make_async_copy(rows1, acc.at[dst_v.at[j]], ssem[b]).wait()

      def body(g, carry):
        j0 = g * NBUF
        for b in range(NBUF):
          scatter1(b, j0 + b)
        for b in range(NBUF):
          scatter1_wait(b, j0 + b)
        return carry

      lax.fori_loop(g0, G, body, 0)

    plsc.subcore_barrier()
    pltpu.sync_copy(acc.at[pl.ds(s * RPT, RPT)],
                    out.at[pl.ds(wid * RPT, RPT)])

  return sc_pass


_sc_deg = _make_sc_pass(16, with_gather=False)
_sc_agg = _make_sc_pass(16, with_gather=True)


def _deg_dis(dacc_ref):
  deg = dacc_ref[:N, 0] + dacc_ref[NROWS:NROWS + N, 0] + 1.0
  return lax.rsqrt(deg)


def _tc_y1(dacc_ref, x_ref, w1_ref, y1_ref):
  dis = _deg_dis(dacc_ref)
  xw = jnp.dot(x_ref[...], w1_ref[...], preferred_element_type=jnp.float32)
  y1_ref[...] = xw * dis[:, None]


def _tc_y2(dacc_ref, acc1_ref, y1_ref, b1_ref, y2_ref):
  dis = _deg_dis(dacc_ref)
  agg = acc1_ref[:N, :] + acc1_ref[NROWS:NROWS + N, :] + y1_ref[...]
  h = jnp.maximum(dis[:, None] * agg + b1_ref[...], 0.0)
  y2_ref[...] = h * dis[:, None]


def _tc_out(dacc_ref, acc2_ref, y2_ref, w2_ref, b2_ref, out_ref):
  dis = _deg_dis(dacc_ref)
  z = dis[:, None] * (acc2_ref[:N, :] + acc2_ref[NROWS:NROWS + N, :] + y2_ref[...])
  o = jnp.dot(z, w2_ref[...], preferred_element_type=jnp.float32) + b2_ref[...]
  m = jnp.max(o, axis=1, keepdims=True)
  zz = o - m
  lse = jnp.log(jnp.sum(jnp.exp(zz), axis=1, keepdims=True))
  out_ref[...] = zz - lse


def kernel(x, edge, W1, b1, W2, b2):
  src3 = edge[0].astype(jnp.int32).reshape(NCHUNKS, K)
  dst3 = edge[1].astype(jnp.int32).reshape(NCHUNKS, K)

  zeros16 = jnp.zeros((RPT, 16), jnp.float32)
  ones16 = jnp.ones((K, 16), jnp.float32)

  dacc = _sc_deg(dst3, zeros16, ones16)

  y1 = pl.pallas_call(
      _tc_y1,
      out_shape=jax.ShapeDtypeStruct((N, 16), jnp.float32),
  )(dacc, x, W1)

  acc1 = _sc_agg(y1, src3, dst3, zeros16)

  y2 = pl.pallas_call(
      _tc_y2,
      out_shape=jax.ShapeDtypeStruct((N, 16), jnp.float32),
  )(dacc, acc1, y1, b1)

  acc2 = _sc_agg(y2, src3, dst3, zeros16)

  out = pl.pallas_call(
      _tc_out,
      out_shape=jax.ShapeDtypeStruct((N, 40), jnp.float32),
  )(dacc, acc2, y2, W2, b2)

  return out


# speedup vs baseline: 53.8881x; 1.0193x over previous
import functools

import jax
import jax.numpy as jnp
from jax import lax
from jax.experimental import pallas as pl
from jax.experimental.pallas import tpu as pltpu
from jax.experimental.pallas import tpu_sc as plsc

N = 10000
E = 320000
NC, NS = 2, 16
NW = NC * NS
K = 128
NCH = 80
NCHUNKS = E // K
G_LAST = (NCHUNKS - (NW - 1) * NCH) // 5
RPT = 626
NROWS = RPT * NS

_mesh = plsc.VectorSubcoreMesh(core_axis_name="c", subcore_axis_name="s")


NBUF = 5
G = NCH // NBUF


def _make_sc_pass(D, with_gather):
  scratch = [
      pltpu.VMEM((NCH, K), jnp.int32),
      pltpu.VMEM_SHARED((NROWS, D), jnp.float32),
  ]
  if with_gather:
    scratch.append(pltpu.VMEM((NCH, K), jnp.int32))
    scratch += [pltpu.VMEM((K, D), jnp.float32) for _ in range(NBUF)]
    scratch += [pltpu.SemaphoreType.DMA for _ in range(2 * NBUF)]
  else:
    scratch.append(pltpu.VMEM((K, D), jnp.float32))
    scratch += [pltpu.SemaphoreType.DMA for _ in range(NBUF)]

  @functools.partial(
      pl.kernel,
      mesh=_mesh,
      out_type=jax.ShapeDtypeStruct((NW * RPT, D), jnp.float32),
      scratch_types=scratch,
      compiler_params=pltpu.CompilerParams(use_tc_tiling_on_sc=False),
  )
  def sc_pass(*refs):
    if with_gather:
      (table, srcidx, dstidx, zeros, out, dst_v, acc, src_v) = refs[:8]
      rows = refs[8:8 + NBUF]
      gsem = refs[8 + NBUF:8 + 2 * NBUF]
      ssem = refs[8 + 2 * NBUF:]
    else:
      (dstidx, zeros, ones, out, dst_v, acc, rows1) = refs[:7]
      ssem = refs[7:]

    c = lax.axis_index("c")
    s = lax.axis_index("s")
    wid = c * NS + s

    base = jnp.minimum(wid * NCH, NCHUNKS - NCH)
    g0 = jnp.where(wid == NW - 1, G - G_LAST, 0)

    pltpu.sync_copy(zeros, acc.at[pl.ds(s * RPT, RPT)])
    pltpu.sync_copy(dstidx.at[pl.ds(base, NCH)], dst_v)
    if with_gather:
      pltpu.sync_copy(srcidx.at[pl.ds(base, NCH)], src_v)
    else:
      pltpu.sync_copy(ones, rows1)
    plsc.subcore_barrier()

    if with_gather:
      def gather(b, j):
        pltpu.async_copy(table.at[src_v.at[j]], rows[b], gsem[b])

      def gather_wait(b, j):
        pltpu.make_async_copy(table.at[src_v.at[j]], rows[b], gsem[b]).wait()

      def scatter(b, j):
        pltpu.async_copy(rows[b], acc.at[dst_v.at[j]], ssem[b], add=True)

      def scatter_wait(b, j):
        pltpu.make_async_copy(rows[b], acc.at[dst_v.at[j]], ssem[b]).wait()

      for b in range(NBUF):
        gather(b, g0 * NBUF + b)

      def body(g, carry):
        j0 = g * NBUF
        for b in range(NBUF):
          gather_wait(b, j0 + b)
          scatter(b, j0 + b)
        for b in range(NBUF):
          scatter_wait(b, j0 + b)
          gather(b, lax.rem(j0 + NBUF + b, NCH))
        return carry

      lax.fori_loop(g0, G, body, 0)
      for b in range(NBUF):
        gather_wait(b, b)
    else:
      def scatter1(b, j):
        pltpu.async_copy(rows1, acc.at[dst_v.at[j]], ssem[b], add=True)

      def scatter1_wait(b, j):
        pltpu.make_async_copy(rows1, acc.at[dst_v.at[j]], ssem[b]).wait()

      def body(g, carry):
        j0 = g * NBUF
        for b in range(NBUF):
          scatter1(b, j0 + b)
        for b in range(NBUF):
          scatter1_wait(b, j0 + b)
        return carry

      lax.fori_loop(g0, G, body, 0)

    plsc.subcore_barrier()
    pltpu.sync_copy(acc.at[pl.ds(s * RPT, RPT)],
                    out.at[pl.ds(wid * RPT, RPT)])

  return sc_pass


_sc_deg = _make_sc_pass(16, with_gather=False)
_sc_agg = _make_sc_pass(16, with_gather=True)


def _deg_dis(dacc_ref):
  deg = dacc_ref[:N, 0] + dacc_ref[NROWS:NROWS + N, 0] + 1.0
  return lax.rsqrt(deg)


def _tc_xw(x_ref, w1_ref, xw_ref):
  xw_ref[...] = jnp.dot(x_ref[...], w1_ref[...],
                        preferred_element_type=jnp.float32)


def _tc_y1(dacc_ref, xw_ref, y1_ref):
  dis = _deg_dis(dacc_ref)
  y1_ref[...] = xw_ref[...] * dis[:, None]


def _tc_y2(dacc_ref, acc1_ref, y1_ref, b1_ref, y2_ref):
  dis = _deg_dis(dacc_ref)
  agg = acc1_ref[:N, :] + acc1_ref[NROWS:NROWS + N, :] + y1_ref[...]
  h = jnp.maximum(dis[:, None] * agg + b1_ref[...], 0.0)
  y2_ref[...] = h * dis[:, None]


def _tc_out(dacc_ref, acc2_ref, y2_ref, w2_ref, b2_ref, out_ref):
  dis = _deg_dis(dacc_ref)
  z = dis[:, None] * (acc2_ref[:N, :] + acc2_ref[NROWS:NROWS + N, :] + y2_ref[...])
  o = jnp.dot(z, w2_ref[...], preferred_element_type=jnp.float32) + b2_ref[...]
  m = jnp.max(o, axis=1, keepdims=True)
  zz = o - m
  lse = jnp.log(jnp.sum(jnp.exp(zz), axis=1, keepdims=True))
  out_ref[...] = zz - lse


def kernel(x, edge, W1, b1, W2, b2):
  src3 = edge[0].astype(jnp.int32).reshape(NCHUNKS, K)
  dst3 = edge[1].astype(jnp.int32).reshape(NCHUNKS, K)

  zeros16 = jnp.zeros((RPT, 16), jnp.float32)
  ones16 = jnp.ones((K, 16), jnp.float32)

  xw = pl.pallas_call(
      _tc_xw,
      out_shape=jax.ShapeDtypeStruct((N, 16), jnp.float32),
  )(x, W1)

  dacc = _sc_deg(dst3, zeros16, ones16)

  y1 = pl.pallas_call(
      _tc_y1,
      out_shape=jax.ShapeDtypeStruct((N, 16), jnp.float32),
  )(dacc, xw)

  acc1 = _sc_agg(y1, src3, dst3, zeros16)

  y2 = pl.pallas_call(
      _tc_y2,
      out_shape=jax.ShapeDtypeStruct((N, 16), jnp.float32),
  )(dacc, acc1, y1, b1)

  acc2 = _sc_agg(y2, src3, dst3, zeros16)

  out = pl.pallas_call(
      _tc_out,
      out_shape=jax.ShapeDtypeStruct((N, 40), jnp.float32),
  )(dacc, acc2, y2, W2, b2)

  return out


# speedup vs baseline: 55.9985x; 1.0392x over previous
import functools

import jax
import jax.numpy as jnp
from jax import lax
from jax.experimental import pallas as pl
from jax.experimental.pallas import tpu as pltpu
from jax.experimental.pallas import tpu_sc as plsc

N = 10000
E = 320000
NC, NS = 2, 16
NW = NC * NS
K = 128
NCH = 80
NCHUNKS = E // K
G_LAST = (NCHUNKS - (NW - 1) * NCH) // 10
RPT = 626
NROWS = RPT * NS

_mesh = plsc.VectorSubcoreMesh(core_axis_name="c", subcore_axis_name="s")


NBUF = 10
G = NCH // NBUF


def _make_sc_pass(D, with_gather):
  scratch = [
      pltpu.VMEM((NCH, K), jnp.int32),
      pltpu.VMEM_SHARED((NROWS, D), jnp.float32),
  ]
  if with_gather:
    scratch.append(pltpu.VMEM((NCH, K), jnp.int32))
    scratch += [pltpu.VMEM((K, D), jnp.float32) for _ in range(NBUF)]
    scratch += [pltpu.SemaphoreType.DMA for _ in range(2 * NBUF)]
  else:
    scratch.append(pltpu.VMEM((K, D), jnp.float32))
    scratch += [pltpu.SemaphoreType.DMA for _ in range(NBUF)]

  @functools.partial(
      pl.kernel,
      mesh=_mesh,
      out_type=jax.ShapeDtypeStruct((NW * RPT, D), jnp.float32),
      scratch_types=scratch,
      compiler_params=pltpu.CompilerParams(use_tc_tiling_on_sc=False),
  )
  def sc_pass(*refs):
    if with_gather:
      (table, srcidx, dstidx, zeros, out, dst_v, acc, src_v) = refs[:8]
      rows = refs[8:8 + NBUF]
      gsem = refs[8 + NBUF:8 + 2 * NBUF]
      ssem = refs[8 + 2 * NBUF:]
    else:
      (dstidx, zeros, ones, out, dst_v, acc, rows1) = refs[:7]
      ssem = refs[7:]

    c = lax.axis_index("c")
    s = lax.axis_index("s")
    wid = c * NS + s

    base = jnp.minimum(wid * NCH, NCHUNKS - NCH)
    g0 = jnp.where(wid == NW - 1, G - G_LAST, 0)

    pltpu.sync_copy(zeros, acc.at[pl.ds(s * RPT, RPT)])
    pltpu.sync_copy(dstidx.at[pl.ds(base, NCH)], dst_v)
    if with_gather:
      pltpu.sync_copy(srcidx.at[pl.ds(base, NCH)], src_v)
    else:
      pltpu.sync_copy(ones, rows1)
    plsc.subcore_barrier()

    if with_gather:
      def gather(b, j):
        pltpu.async_copy(table.at[src_v.at[j]], rows[b], gsem[b])

      def gather_wait(b, j):
        pltpu.make_async_copy(table.at[src_v.at[j]], rows[b], gsem[b]).wait()

      def scatter(b, j):
        pltpu.async_copy(rows[b], acc.at[dst_v.at[j]], ssem[b], add=True)

      def scatter_wait(b, j):
        pltpu.make_async_copy(rows[b], acc.at[dst_v.at[j]], ssem[b]).wait()

      for b in range(NBUF):
        gather(b, g0 * NBUF + b)

      def body(g, carry):
        j0 = g * NBUF
        for b in range(NBUF):
          gather_wait(b, j0 + b)
          scatter(b, j0 + b)
        for b in range(NBUF):
          scatter_wait(b, j0 + b)
          gather(b, lax.rem(j0 + NBUF + b, NCH))
        return carry

      lax.fori_loop(g0, G, body, 0)
      for b in range(NBUF):
        gather_wait(b, b)
    else:
      def scatter1(b, j):
        pltpu.async_copy(rows1, acc.at[dst_v.at[j]], ssem[b], add=True)

      def scatter1_wait(b, j):
        pltpu.make_async_copy(rows1, acc.at[dst_v.at[j]], ssem[b]).wait()

      def body(g, carry):
        j0 = g * NBUF
        for b in range(NBUF):
          scatter1(b, j0 + b)
        for b in range(NBUF):
          scatter1_wait(b, j0 + b)
        return carry

      lax.fori_loop(g0, G, body, 0)

    plsc.subcore_barrier()
    pltpu.sync_copy(acc.at[pl.ds(s * RPT, RPT)],
                    out.at[pl.ds(wid * RPT, RPT)])

  return sc_pass


_sc_deg = _make_sc_pass(16, with_gather=False)
_sc_agg = _make_sc_pass(16, with_gather=True)


def _deg_dis(dacc_ref):
  deg = dacc_ref[:N, 0] + dacc_ref[NROWS:NROWS + N, 0] + 1.0
  return lax.rsqrt(deg)


def _tc_xw(x_ref, w1_ref, xw_ref):
  xw_ref[...] = jnp.dot(x_ref[...], w1_ref[...],
                        preferred_element_type=jnp.float32)


def _tc_y1(dacc_ref, xw_ref, y1_ref):
  dis = _deg_dis(dacc_ref)
  y1_ref[...] = xw_ref[...] * dis[:, None]


def _tc_y2(dacc_ref, acc1_ref, y1_ref, b1_ref, y2_ref):
  dis = _deg_dis(dacc_ref)
  agg = acc1_ref[:N, :] + acc1_ref[NROWS:NROWS + N, :] + y1_ref[...]
  h = jnp.maximum(dis[:, None] * agg + b1_ref[...], 0.0)
  y2_ref[...] = h * dis[:, None]


def _tc_out(dacc_ref, acc2_ref, y2_ref, w2_ref, b2_ref, out_ref):
  dis = _deg_dis(dacc_ref)
  z = dis[:, None] * (acc2_ref[:N, :] + acc2_ref[NROWS:NROWS + N, :] + y2_ref[...])
  o = jnp.dot(z, w2_ref[...], preferred_element_type=jnp.float32) + b2_ref[...]
  m = jnp.max(o, axis=1, keepdims=True)
  zz = o - m
  lse = jnp.log(jnp.sum(jnp.exp(zz), axis=1, keepdims=True))
  out_ref[...] = zz - lse


def kernel(x, edge, W1, b1, W2, b2):
  src3 = edge[0].astype(jnp.int32).reshape(NCHUNKS, K)
  dst3 = edge[1].astype(jnp.int32).reshape(NCHUNKS, K)

  zeros16 = jnp.zeros((RPT, 16), jnp.float32)
  ones16 = jnp.ones((K, 16), jnp.float32)

  xw = pl.pallas_call(
      _tc_xw,
      out_shape=jax.ShapeDtypeStruct((N, 16), jnp.float32),
  )(x, W1)

  dacc = _sc_deg(dst3, zeros16, ones16)

  y1 = pl.pallas_call(
      _tc_y1,
      out_shape=jax.ShapeDtypeStruct((N, 16), jnp.float32),
  )(dacc, xw)

  acc1 = _sc_agg(y1, src3, dst3, zeros16)

  y2 = pl.pallas_call(
      _tc_y2,
      out_shape=jax.ShapeDtypeStruct((N, 16), jnp.float32),
  )(dacc, acc1, y1, b1)

  acc2 = _sc_agg(y2, src3, dst3, zeros16)

  out = pl.pallas_call(
      _tc_out,
      out_shape=jax.ShapeDtypeStruct((N, 40), jnp.float32),
  )(dacc, acc2, y2, W2, b2)

  return out
